# baseline (speedup 1.0000x reference)
import functools

import jax
import jax.numpy as jnp
from jax import lax
from jax.experimental import pallas as pl
from jax.experimental.pallas import tpu as pltpu
from jax.experimental.pallas import tpu_sc as plsc

N_NODES = 10000
N_EDGES = 320000
D_FEAT = 128

NC = 2
NS = 16
NW = NC * NS
EW = N_EDGES // NW

C = 400
NCHUNK = EW // C
SUB = 80
NSUB = C // SUB


def _dot_kernel(src_idx, dst_idx, xsrc, xdst, out, siv, div, ra, rb, ov,
                sem_a, sem_b):
    wid = lax.axis_index("s") * NC + lax.axis_index("c")
    base = wid * EW

    def chunk_body(i, carry):
        cb = base + i * C
        pltpu.sync_copy(src_idx.at[pl.ds(cb, C)], siv)
        pltpu.sync_copy(dst_idx.at[pl.ds(cb, C)], div)
        copies = []
        for j in range(NSUB):
            copies.append(
                pltpu.async_copy(xsrc.at[siv.at[pl.ds(j * SUB, SUB)]],
                                 ra.at[pl.ds(j * SUB, SUB)], sem_a))
            copies.append(
                pltpu.async_copy(xdst.at[div.at[pl.ds(j * SUB, SUB)]],
                                 rb.at[pl.ds(j * SUB, SUB)], sem_b))
        for cp in copies:
            cp.wait()

        lane = lax.iota(jnp.int32, 16)

        def group_body(g, gcarry):
            e_vec = g * 16 + lane
            dv = jnp.zeros((16,), jnp.int32)
            acc0 = jnp.zeros((16,), jnp.float32)
            acc1 = jnp.zeros((16,), jnp.float32)
            acc2 = jnp.zeros((16,), jnp.float32)
            acc3 = jnp.zeros((16,), jnp.float32)
            for d in range(0, D_FEAT, 4):
                acc0 = acc0 + (plsc.load_gather(ra, [e_vec, dv]) *
                               plsc.load_gather(rb, [e_vec, dv]))
                dv1 = dv + 1
                acc1 = acc1 + (plsc.load_gather(ra, [e_vec, dv1]) *
                               plsc.load_gather(rb, [e_vec, dv1]))
                dv2 = dv + 2
                acc2 = acc2 + (plsc.load_gather(ra, [e_vec, dv2]) *
                               plsc.load_gather(rb, [e_vec, dv2]))
                dv3 = dv + 3
                acc3 = acc3 + (plsc.load_gather(ra, [e_vec, dv3]) *
                               plsc.load_gather(rb, [e_vec, dv3]))
                dv = dv + 4
            ov[pl.ds(g * 16, 16)] = (acc0 + acc1) + (acc2 + acc3)
            return gcarry

        lax.fori_loop(0, C // 16, group_body, 0)
        pltpu.sync_copy(ov, out.at[pl.ds(cb, C)])
        return carry

    lax.fori_loop(0, NCHUNK, chunk_body, 0)


_mesh = plsc.VectorSubcoreMesh(core_axis_name="c", subcore_axis_name="s")

_kernel_call = functools.partial(
    pl.kernel,
    mesh=_mesh,
    compiler_params=pltpu.CompilerParams(needs_layout_passes=False),
    out_type=jax.ShapeDtypeStruct((N_EDGES,), jnp.float32),
    scratch_types=[
        pltpu.VMEM((C,), jnp.int32),
        pltpu.VMEM((C,), jnp.int32),
        pltpu.VMEM((C, D_FEAT), jnp.float32),
        pltpu.VMEM((C, D_FEAT), jnp.float32),
        pltpu.VMEM((C,), jnp.float32),
        pltpu.SemaphoreType.DMA,
        pltpu.SemaphoreType.DMA,
    ],
)(_dot_kernel)


@jax.jit
def kernel(x_src, x_dst, edge_label_index):
    eli = edge_label_index.astype(jnp.int32)
    return _kernel_call(eli[0], eli[1], x_src, x_dst)


# speedup vs baseline: 4.4615x; 4.4615x over previous
import functools

import jax
import jax.numpy as jnp
from jax import lax
from jax.experimental import pallas as pl
from jax.experimental.pallas import tpu as pltpu
from jax.experimental.pallas import tpu_sc as plsc

N_NODES = 10000
N_EDGES = 320000
D_FEAT = 128

NC = 2
NS = 16
NW = NC * NS
EW = N_EDGES // NW

C = 80
NCHUNK = EW // C
NPAIR = NCHUNK // 2


def _dot_kernel(src_idx, dst_idx, xsrc, xdst, out,
                siv, div, ra0, rb0, ra1, rb1, ov0, ov1,
                sem0, sem1, semo0, semo1):
    wid = lax.axis_index("s") * NC + lax.axis_index("c")
    base = wid * EW

    pltpu.sync_copy(src_idx.at[wid], siv)
    pltpu.sync_copy(dst_idx.at[wid], div)

    def fire(chunk, ra, rb, sem):
        pltpu.async_copy(xsrc.at[siv.at[chunk]], ra, sem)
        pltpu.async_copy(xdst.at[div.at[chunk]], rb, sem)

    def drain(ra, rb, sem):
        pltpu.make_async_copy(xsrc.at[siv.at[0]], ra, sem).wait()
        pltpu.make_async_copy(xdst.at[div.at[0]], rb, sem).wait()

    zero16 = jnp.zeros((16,), jnp.float32)

    def compute(ra, rb, ov):
        for k in range(C // 16):
            ov[pl.ds(k * 16, 16)] = zero16

        def group_body(g, gcarry):
            for u in range(16):
                e = g * 16 + u
                acc0 = ra[e, pl.ds(0, 16)] * rb[e, pl.ds(0, 16)]
                acc1 = ra[e, pl.ds(16, 16)] * rb[e, pl.ds(16, 16)]
                acc2 = ra[e, pl.ds(32, 16)] * rb[e, pl.ds(32, 16)]
                acc3 = ra[e, pl.ds(48, 16)] * rb[e, pl.ds(48, 16)]
                acc0 = acc0 + ra[e, pl.ds(64, 16)] * rb[e, pl.ds(64, 16)]
                acc1 = acc1 + ra[e, pl.ds(80, 16)] * rb[e, pl.ds(80, 16)]
                acc2 = acc2 + ra[e, pl.ds(96, 16)] * rb[e, pl.ds(96, 16)]
                acc3 = acc3 + ra[e, pl.ds(112, 16)] * rb[e, pl.ds(112, 16)]
                acc = (acc0 + acc1) + (acc2 + acc3)
                plsc.addupdate_scatter(ov, [jnp.full((16,), e, jnp.int32)],
                                       acc)
            return gcarry

        lax.fori_loop(0, C // 16, group_body, 0)

    def wait_out(ov, semo):
        pltpu.make_async_copy(ov, out.at[pl.ds(base, C)], semo).wait()

    fire(0, ra0, rb0, sem0)
    fire(1, ra1, rb1, sem1)

    def pair_body(p, carry):
        drain(ra0, rb0, sem0)

        @pl.when(p > 0)
        def _():
            wait_out(ov0, semo0)

        compute(ra0, rb0, ov0)
        fire(2 * p + 2, ra0, rb0, sem0)
        pltpu.async_copy(ov0, out.at[pl.ds(base + (2 * p) * C, C)], semo0)

        drain(ra1, rb1, sem1)

        @pl.when(p > 0)
        def _():
            wait_out(ov1, semo1)

        compute(ra1, rb1, ov1)

        @pl.when(p < NPAIR - 1)
        def _():
            fire(2 * p + 3, ra1, rb1, sem1)

        pltpu.async_copy(ov1, out.at[pl.ds(base + (2 * p + 1) * C, C)], semo1)
        return carry

    lax.fori_loop(0, NPAIR, pair_body, 0)

    drain(ra0, rb0, sem0)
    wait_out(ov0, semo0)
    compute(ra0, rb0, ov0)
    pltpu.async_copy(ov0, out.at[pl.ds(base + (NCHUNK - 1) * C, C)], semo0)
    wait_out(ov0, semo0)
    wait_out(ov1, semo1)


_mesh = plsc.VectorSubcoreMesh(core_axis_name="c", subcore_axis_name="s")

_kernel_call = functools.partial(
    pl.kernel,
    mesh=_mesh,
    compiler_params=pltpu.CompilerParams(needs_layout_passes=False),
    out_type=jax.ShapeDtypeStruct((N_EDGES,), jnp.float32),
    scratch_types=[
        pltpu.VMEM((NCHUNK, C), jnp.int32),
        pltpu.VMEM((NCHUNK, C), jnp.int32),
        pltpu.VMEM((C, D_FEAT), jnp.float32),
        pltpu.VMEM((C, D_FEAT), jnp.float32),
        pltpu.VMEM((C, D_FEAT), jnp.float32),
        pltpu.VMEM((C, D_FEAT), jnp.float32),
        pltpu.VMEM((C,), jnp.float32),
        pltpu.VMEM((C,), jnp.float32),
        pltpu.SemaphoreType.DMA,
        pltpu.SemaphoreType.DMA,
        pltpu.SemaphoreType.DMA,
        pltpu.SemaphoreType.DMA,
    ],
)(_dot_kernel)


@jax.jit
def kernel(x_src, x_dst, edge_label_index):
    eli = edge_label_index.astype(jnp.int32)
    src3 = eli[0].reshape(NW, NCHUNK, C)
    dst3 = eli[1].reshape(NW, NCHUNK, C)
    return _kernel_call(src3, dst3, x_src, x_dst)
